# Initial kernel scaffold; baseline (speedup 1.0000x reference)
import jax
import jax.numpy as jnp
from jax.experimental import pallas as pl


def kernel(pos_u, pos_v, neg_u, neg_v, u_emb, v_emb):
    raise NotImplementedError("write your pallas kernel here")


# speedup vs baseline: 32.2034x; 32.2034x over previous
import functools

import jax
import jax.numpy as jnp
from jax import lax
from jax.experimental import pallas as pl
from jax.experimental.pallas import tpu as pltpu
from jax.experimental.pallas import tpu_sc as plsc

EMB_DIM = 64
CTX = 10
B_POS = 16384
B_NEG = 81920
B_TOT = B_POS + B_NEG
NC = 2
NS = 16
NW = NC * NS
ITEMS_PER_W = B_TOT // NW
C = 48
NCHUNK = ITEMS_PER_W // C
NBUF = 2
IDXG = 120
NGATHER = (C * CTX) // IDXG
R16 = B_TOT // 16


def _sc_body(u_emb, v_emb, all_u, all_v, out,
             idx_u_all, idx_v_all, rows_u0, rows_u1, rows_v0, rows_v1,
             parts0, parts1, semg0, semg1, semo0, semo1):
    rows_u = (rows_u0, rows_u1)
    rows_v = (rows_v0, rows_v1)
    parts = (parts0, parts1)
    semg = (semg0, semg1)
    semo = (semo0, semo1)

    wid = lax.axis_index("s") * NC + lax.axis_index("c")
    base = wid * ITEMS_PER_W
    pltpu.sync_copy(all_u.at[pl.ds(base, ITEMS_PER_W)], idx_u_all)
    pltpu.sync_copy(all_v.at[pl.ds(base * CTX, ITEMS_PER_W * CTX)], idx_v_all)

    def issue(j, b):
        pltpu.async_copy(
            v_emb.at[idx_u_all.at[pl.ds(j * C, C)]], rows_u[b], semg[b])
        for k in range(NGATHER):
            pltpu.async_copy(
                u_emb.at[idx_v_all.at[pl.ds(j * (C * CTX) + k * IDXG, IDXG)]],
                rows_v[b].at[pl.ds(k * IDXG, IDXG)], semg[b])

    def drain_gathers(b):
        pltpu.make_async_copy(v_emb.at[pl.ds(0, C)], rows_u[b], semg[b]).wait()
        pltpu.make_async_copy(
            u_emb.at[pl.ds(0, C * CTX)], rows_v[b], semg[b]).wait()

    def compute(j, b):
        rv, ru, pt = rows_v[b], rows_u[b], parts[b]

        def item_body(i, carry):
            r = i * CTX
            accs = [rv[r, pl.ds(d * 16, 16)] for d in range(4)]
            for c in range(1, CTX):
                for d in range(4):
                    accs[d] = accs[d] + rv[r + c, pl.ds(d * 16, 16)]
            part = accs[0] * ru[i, pl.ds(0, 16)]
            for d in range(1, 4):
                part = part + accs[d] * ru[i, pl.ds(d * 16, 16)]
            pt[i, :] = part
            return carry

        lax.fori_loop(0, C, item_body, 0)

    issue(0, 0)

    def outer(g, carry):
        for b in range(NBUF):
            j = g * NBUF + b
            jn = j + 1

            @pl.when(jn < NCHUNK)
            def _():
                issue(jn, b ^ 1)

            drain_gathers(b)

            @pl.when(j >= NBUF)
            def _():
                pltpu.make_async_copy(
                    parts[b], out.at[pl.ds(0, C)], semo[b]).wait()

            compute(j, b)
            pltpu.async_copy(parts[b], out.at[pl.ds(base + j * C, C)], semo[b])
        return carry

    lax.fori_loop(0, NCHUNK // NBUF, outer, 0)
    for b in range(NBUF):
        pltpu.make_async_copy(parts[b], out.at[pl.ds(0, C)], semo[b]).wait()


_sc_scores = functools.partial(
    pl.kernel,
    out_type=jax.ShapeDtypeStruct((B_TOT, 16), jnp.float32),
    mesh=plsc.VectorSubcoreMesh(core_axis_name="c", subcore_axis_name="s"),
    scratch_types=[
        pltpu.VMEM((ITEMS_PER_W,), jnp.int32),
        pltpu.VMEM((ITEMS_PER_W * CTX,), jnp.int32),
        pltpu.VMEM((C, EMB_DIM), jnp.float32),
        pltpu.VMEM((C, EMB_DIM), jnp.float32),
        pltpu.VMEM((C * CTX, EMB_DIM), jnp.float32),
        pltpu.VMEM((C * CTX, EMB_DIM), jnp.float32),
        pltpu.VMEM((C, 16), jnp.float32),
        pltpu.VMEM((C, 16), jnp.float32),
        pltpu.SemaphoreType.DMA,
        pltpu.SemaphoreType.DMA,
        pltpu.SemaphoreType.DMA,
        pltpu.SemaphoreType.DMA,
    ],
    compiler_params=pltpu.CompilerParams(use_tc_tiling_on_sc=False),
)(_sc_body)


def _tc_body(parts_ref, o_ref):
    x = parts_ref[...]
    sel = (lax.broadcasted_iota(jnp.int32, (256, 16), 0) // 16
           == lax.broadcasted_iota(jnp.int32, (256, 16), 1)).astype(jnp.float32)
    s = jnp.dot(x, sel, preferred_element_type=jnp.float32)
    row = lax.broadcasted_iota(jnp.int32, (R16, 16), 0)
    sign = jnp.where(row < B_POS // 16, 1.0, -1.0)
    t = s * sign
    ls = jnp.minimum(t, 0.0) - jnp.log(1.0 + jnp.exp(-jnp.abs(t)))
    o_ref[0, 0] = -jnp.sum(ls)


_tc_finish = pl.pallas_call(
    _tc_body,
    out_shape=jax.ShapeDtypeStruct((1, 1), jnp.float32),
    out_specs=pl.BlockSpec(memory_space=pltpu.SMEM),
)


def kernel(pos_u, pos_v, neg_u, neg_v, u_emb, v_emb):
    all_u = jnp.concatenate([pos_u, neg_u]).astype(jnp.int32)
    all_v = jnp.concatenate(
        [pos_v.reshape(-1), neg_v.reshape(-1)]).astype(jnp.int32)
    parts = _sc_scores(u_emb, v_emb, all_u, all_v)
    loss = _tc_finish(parts.reshape(R16, 256))
    return loss[0, 0]
